# baseline (speedup 1.0000x reference)
import jax
import jax.numpy as jnp
from jax.experimental import pallas as pl

S = 16
N = 128
P = 8
KNN = 4
C = 64
H = 4
DH = C // H

SCN = 4
NL = SCN * N

_INTERPRET = False

_SELU_ALPHA = 1.6732632423543772
_SELU_SCALE = 1.0507009873554805


def _selu(v):
    neg = _SELU_ALPHA * (jnp.exp(v) - 1.0)
    return _SELU_SCALE * jnp.where(v > 0, v, neg)


def _ln(v, g, b):
    mu = jnp.mean(v, axis=-1, keepdims=True)
    var = jnp.var(v, axis=-1, keepdims=True)
    return (v - mu) / jnp.sqrt(var + 1e-5) * g + b


def _lnT(v, g, b):
    mu = jnp.mean(v, axis=0, keepdims=True)
    var = jnp.mean(v * v, axis=0, keepdims=True) - mu * mu
    return (v - mu) * jax.lax.rsqrt(var + 1e-5) * g + b


def _knn_pen(d3):
    ib = jax.lax.broadcasted_iota(jnp.int32, (P, P, NL), 1)
    cnt = jnp.zeros((P, P, NL), jnp.float32)
    for s in range(1, P):
        oth = jnp.concatenate([d3[:, s:, :], d3[:, :s, :]], axis=1)
        win = (oth < d3) | ((oth == d3) & (ib >= P - s))
        cnt += win.astype(jnp.float32)
    return jnp.where(cnt < float(KNN), 0.0, -1e9).reshape(P * P, NL)


def _convT(xs, WT, cc, g, b):
    tcols, dmap = {}, {}
    zrow = jnp.zeros((1, NL), jnp.float32)
    for a in range(P):
        for bb in range(P):
            t = xs[bb] - xs[a]
            tcols[(a, bb)] = t
            if a == bb:
                dmap[(a, bb)] = zrow
            elif a < bb:
                dmap[(a, bb)] = jnp.sum(t * t, axis=0, keepdims=True)
            else:
                dmap[(a, bb)] = dmap[(bb, a)]
    d3 = jnp.concatenate(
        [dmap[(a, bb)] for a in range(P) for bb in range(P)],
        axis=0).reshape(P, P, NL)
    pen = _knn_pen(d3)
    out = []
    for a in range(P):
        eTa = jnp.concatenate(
            [jnp.concatenate([tcols[(a, bb)], xs[a]], axis=0)
             for bb in range(P)], axis=1)
        pre = WT @ eTa + cc
        penrow = jnp.concatenate(
            [pen[a * P + bb:a * P + bb + 1, :] for bb in range(P)], axis=1)
        y = _selu(_lnT(pre, g, b)) + penrow
        acc = y[:, :NL]
        for bb in range(1, P):
            acc = jnp.maximum(acc, y[:, bb * NL:(bb + 1) * NL])
        out.append(acc)
    return out


def _attn2(p12, QB, qb, KB, kb, VB, vb, OB, ob, g1, b1, g2, b2):
    q = p12 @ QB + qb
    k = p12 @ KB + kb
    v = p12 @ VB + vb
    per_scene = []
    for si in range(SCN):
        rows = slice(si * N, (si + 1) * N)
        outs = []
        for h in range(2 * H):
            sl = slice(DH * h, DH * (h + 1))
            qh, kh, vh = q[rows, sl], k[rows, sl], v[rows, sl]
            lg = jax.lax.dot_general(
                qh, kh, (((1,), (1,)), ((), ())),
                preferred_element_type=jnp.float32) * (1.0 / (DH ** 0.5))
            ex = jnp.exp(lg - jnp.max(lg, axis=-1, keepdims=True))
            pr = ex * (1.0 / jnp.sum(ex, axis=-1, keepdims=True))
            outs.append(pr @ vh)
        per_scene.append(jnp.concatenate(outs, axis=1))
    o = jnp.concatenate(per_scene, axis=0)
    res = p12 + (o @ OB + ob)
    ln12 = jnp.concatenate(
        [_ln(res[:, :C], g1, b1), _ln(res[:, C:], g2, b2)], axis=1)
    return _selu(ln12)


def _pmaxT(chunks):
    acc = chunks[0]
    for t in chunks[1:]:
        acc = jnp.maximum(acc, t)
    return acc.T


def _scene_kernel(xt_ref,
                  W1T_ref, c1_ref, g1_ref, h1_ref,
                  W2T_ref, c2_ref, g2_ref, h2_ref,
                  QB_ref, qb_ref, KB_ref, kb_ref, VB_ref, vb_ref,
                  OB_ref, ob_ref, lg1_ref, lb1_ref, lg2_ref, lb2_ref,
                  Wp_ref, cp_ref, gp_ref, hp_ref,
                  out_ref):
    xs = [xt_ref[0, p_] for p_ in range(P)]

    x1 = _convT(xs, W1T_ref[...], c1_ref[...], g1_ref[...], h1_ref[...])
    p1 = _pmaxT(x1)

    x2 = _convT(x1, W2T_ref[...], c2_ref[...], g2_ref[...], h2_ref[...])
    p2 = _pmaxT(x2)

    p12 = jnp.concatenate([p1, p2], axis=1)
    a12 = _attn2(p12, QB_ref[...], qb_ref[...], KB_ref[...], kb_ref[...],
                 VB_ref[...], vb_ref[...], OB_ref[...], ob_ref[...],
                 lg1_ref[...], lb1_ref[...], lg2_ref[...], lb2_ref[...])

    z = a12 @ Wp_ref[...] + cp_ref[...]
    out_ref[...] = _selu(_ln(z, gp_ref[...], hp_ref[...]))


def _full(shape):
    nd = len(shape)
    return pl.BlockSpec(shape, lambda s, _nd=nd: (0,) * _nd)


def kernel(x, n_nodes, W1, c1, g1, h1, W2, c2, g2, h2,
           Q1, qb1, K1, kb1, V1, vb1, O1, ob1, lg1, lb1,
           Q2, qb2, K2, kb2, V2, vb2, O2, ob2, lg2, lb2,
           Wp, cp, gp, hp):
    del n_nodes
    row = lambda v: v.reshape(1, C)
    col = lambda v: v.reshape(C, 1)
    xr = jnp.pad(x.reshape(S // SCN, SCN, N, P, 3),
                 ((0, 0), (0, 0), (0, 0), (0, 0), (0, 5)))
    xt = xr.transpose(0, 3, 4, 1, 2).reshape(S // SCN, P, 8, NL)
    z35 = jnp.zeros((C, 5), jnp.float32)
    W1Tp = jnp.concatenate([W1[:3].T, z35, W1[3:].T, z35], axis=1)

    zc = jnp.zeros((C, C), jnp.float32)
    bd = lambda A, B: jnp.concatenate(
        [jnp.concatenate([A, zc], axis=1), jnp.concatenate([zc, B], axis=1)],
        axis=0)
    cat2 = lambda u, v: jnp.concatenate([u, v]).reshape(1, 2 * C)

    ops = [xt,
           W1Tp, col(c1), col(g1), col(h1),
           W2.T, col(c2), col(g2), col(h2),
           bd(Q1, Q2), cat2(qb1, qb2), bd(K1, K2), cat2(kb1, kb2),
           bd(V1, V2), cat2(vb1, vb2), bd(O1, O2), cat2(ob1, ob2),
           row(lg1), row(lb1), row(lg2), row(lb2),
           Wp, row(cp), row(gp), row(hp)]

    in_specs = [pl.BlockSpec((1, P, 8, NL), lambda s: (s, 0, 0, 0))]
    in_specs += [_full(op.shape) for op in ops[1:]]

    return pl.pallas_call(
        _scene_kernel,
        grid=(S // SCN,),
        in_specs=in_specs,
        out_specs=pl.BlockSpec((NL, C), lambda s: (s, 0)),
        out_shape=jax.ShapeDtypeStruct((S * N, C), jnp.float32),
        interpret=_INTERPRET,
    )(*ops)


# speedup vs baseline: 1.0698x; 1.0698x over previous
import jax
import jax.numpy as jnp
from jax.experimental import pallas as pl

S = 16
N = 128
P = 8
KNN = 4
C = 64
H = 4
DH = C // H

SCN = 4
NL = SCN * N

_INTERPRET = False

_SELU_ALPHA = 1.6732632423543772
_SELU_SCALE = 1.0507009873554805


def _selu(v):
    neg = _SELU_ALPHA * (jnp.exp(v) - 1.0)
    return _SELU_SCALE * jnp.where(v > 0, v, neg)


def _ln(v, g, b):
    mu = jnp.mean(v, axis=-1, keepdims=True)
    var = jnp.var(v, axis=-1, keepdims=True)
    return (v - mu) / jnp.sqrt(var + 1e-5) * g + b


def _lnT(v, g, b):
    mu = jnp.mean(v, axis=0, keepdims=True)
    var = jnp.mean(v * v, axis=0, keepdims=True) - mu * mu
    return (v - mu) * jax.lax.rsqrt(var + 1e-5) * g + b


def _knn_pen(d3):
    ib = jax.lax.broadcasted_iota(jnp.int32, (P, P, NL), 1)
    cnt = jnp.zeros((P, P, NL), jnp.float32)
    for s in range(1, P):
        oth = jnp.concatenate([d3[:, s:, :], d3[:, :s, :]], axis=1)
        win = (oth < d3) | ((oth == d3) & (ib >= P - s))
        cnt += win.astype(jnp.float32)
    return jnp.where(cnt < float(KNN), 0.0, -1e9).reshape(P * P, NL)


def _convT(xs, WT, cc, g, b):
    tcols, dmap = {}, {}
    zrow = jnp.zeros((1, NL), jnp.float32)
    for a in range(P):
        for bb in range(P):
            t = xs[bb] - xs[a]
            tcols[(a, bb)] = t
            if a == bb:
                dmap[(a, bb)] = zrow
            elif a < bb:
                dmap[(a, bb)] = jnp.sum(t * t, axis=0, keepdims=True)
            else:
                dmap[(a, bb)] = dmap[(bb, a)]
    d3 = jnp.concatenate(
        [dmap[(a, bb)] for a in range(P) for bb in range(P)],
        axis=0).reshape(P, P, NL)
    pen = _knn_pen(d3)
    out = []
    for a in range(P):
        eTa = jnp.concatenate(
            [jnp.concatenate([tcols[(a, bb)], xs[a]], axis=0)
             for bb in range(P)], axis=1)
        pre = WT @ eTa + cc
        penrow = jnp.concatenate(
            [pen[a * P + bb:a * P + bb + 1, :] for bb in range(P)], axis=1)
        y = _selu(_lnT(pre, g, b)) + penrow
        acc = y[:, :NL]
        for bb in range(1, P):
            acc = jnp.maximum(acc, y[:, bb * NL:(bb + 1) * NL])
        out.append(acc)
    return out


def _attn(p, Q, qb, K, kb, V, vb, O, ob, g, b):
    q = p @ Q + qb
    k = p @ K + kb
    v = p @ V + vb
    per_scene = []
    for si in range(SCN):
        rows = slice(si * N, (si + 1) * N)
        outs = []
        for h in range(H):
            sl = slice(DH * h, DH * (h + 1))
            qh, kh, vh = q[rows, sl], k[rows, sl], v[rows, sl]
            lg = jax.lax.dot_general(
                qh, kh, (((1,), (1,)), ((), ())),
                preferred_element_type=jnp.float32) * (1.0 / (DH ** 0.5))
            ex = jnp.exp(lg - jnp.max(lg, axis=-1, keepdims=True))
            pr = ex * (1.0 / jnp.sum(ex, axis=-1, keepdims=True))
            outs.append(pr @ vh)
        per_scene.append(jnp.concatenate(outs, axis=1))
    o = jnp.concatenate(per_scene, axis=0)
    return _selu(_ln(p + (o @ O + ob), g, b))


def _pmaxT(chunks):
    acc = chunks[0]
    for t in chunks[1:]:
        acc = jnp.maximum(acc, t)
    return acc.T


def _scene_kernel(xt_ref,
                  W1T_ref, c1_ref, g1_ref, h1_ref,
                  W2T_ref, c2_ref, g2_ref, h2_ref,
                  Q1_ref, qb1_ref, K1_ref, kb1_ref, V1_ref, vb1_ref,
                  O1_ref, ob1_ref, lg1_ref, lb1_ref,
                  Q2_ref, qb2_ref, K2_ref, kb2_ref, V2_ref, vb2_ref,
                  O2_ref, ob2_ref, lg2_ref, lb2_ref,
                  Wpa_ref, Wpb_ref, cp_ref, gp_ref, hp_ref,
                  out_ref):
    xs = [xt_ref[0, p_] for p_ in range(P)]

    x1 = _convT(xs, W1T_ref[...], c1_ref[...], g1_ref[...], h1_ref[...])
    p1 = _pmaxT(x1)

    x2 = _convT(x1, W2T_ref[...], c2_ref[...], g2_ref[...], h2_ref[...])
    p2 = _pmaxT(x2)

    a1 = _attn(p1, Q1_ref[...], qb1_ref[...], K1_ref[...], kb1_ref[...],
               V1_ref[...], vb1_ref[...], O1_ref[...], ob1_ref[...],
               lg1_ref[...], lb1_ref[...])
    a2 = _attn(p2, Q2_ref[...], qb2_ref[...], K2_ref[...], kb2_ref[...],
               V2_ref[...], vb2_ref[...], O2_ref[...], ob2_ref[...],
               lg2_ref[...], lb2_ref[...])

    z = a1 @ Wpa_ref[...] + a2 @ Wpb_ref[...] + cp_ref[...]
    out_ref[...] = _selu(_ln(z, gp_ref[...], hp_ref[...]))


def _full(shape):
    nd = len(shape)
    return pl.BlockSpec(shape, lambda s, _nd=nd: (0,) * _nd)


def kernel(x, n_nodes, W1, c1, g1, h1, W2, c2, g2, h2,
           Q1, qb1, K1, kb1, V1, vb1, O1, ob1, lg1, lb1,
           Q2, qb2, K2, kb2, V2, vb2, O2, ob2, lg2, lb2,
           Wp, cp, gp, hp):
    del n_nodes
    row = lambda v: v.reshape(1, C)
    col = lambda v: v.reshape(C, 1)
    xr = jnp.pad(x.reshape(S // SCN, SCN, N, P, 3),
                 ((0, 0), (0, 0), (0, 0), (0, 0), (0, 5)))
    xt = xr.transpose(0, 3, 4, 1, 2).reshape(S // SCN, P, 8, NL)
    z35 = jnp.zeros((C, 5), jnp.float32)
    W1Tp = jnp.concatenate([W1[:3].T, z35, W1[3:].T, z35], axis=1)

    ops = [xt,
           W1Tp, col(c1), col(g1), col(h1),
           W2.T, col(c2), col(g2), col(h2),
           Q1, row(qb1), K1, row(kb1), V1, row(vb1), O1, row(ob1),
           row(lg1), row(lb1),
           Q2, row(qb2), K2, row(kb2), V2, row(vb2), O2, row(ob2),
           row(lg2), row(lb2),
           Wp[:C], Wp[C:], row(cp), row(gp), row(hp)]

    in_specs = [pl.BlockSpec((1, P, 8, NL), lambda s: (s, 0, 0, 0))]
    in_specs += [_full(op.shape) for op in ops[1:]]

    return pl.pallas_call(
        _scene_kernel,
        grid=(S // SCN,),
        in_specs=in_specs,
        out_specs=pl.BlockSpec((NL, C), lambda s: (s, 0)),
        out_shape=jax.ShapeDtypeStruct((S * N, C), jnp.float32),
        interpret=_INTERPRET,
    )(*ops)
